# baseline (speedup 1.0000x reference)
import functools

import jax
import jax.numpy as jnp
from jax import lax
from jax.experimental import pallas as pl
from jax.experimental.pallas import tpu as pltpu
from jax.experimental.pallas import tpu_sc as plsc

_NEG_T = 0.4
_POS_T = 0.5
_C = 8192

_LEVEL_BASE = (0, 36864, 46080, 48384, 48960)
_LEVEL_STRIDE = (8, 16, 32, 64, 128)
_LEVEL_SHIFT = (6, 5, 4, 3, 2)


def _iou_block(pri_ref, bb_ref):
    pr = pri_ref[:, :]
    pcx, pcy = pr[0:1, :], pr[1:2, :]
    pw, ph = pr[2:3, :], pr[3:4, :]
    px1 = pcx - pw / 2
    py1 = pcy - ph / 2
    px2 = pcx + pw / 2
    py2 = pcy + ph / 2
    area_p = (px2 - px1) * (py2 - py1)

    gx1, gy1 = bb_ref[:, 0:1], bb_ref[:, 1:2]
    gx2, gy2 = bb_ref[:, 2:3], bb_ref[:, 3:4]
    area_g = (gx2 - gx1) * (gy2 - gy1)

    ltx = jnp.maximum(gx1, px1)
    lty = jnp.maximum(gy1, py1)
    rbx = jnp.minimum(gx2, px2)
    rby = jnp.minimum(gy2, py2)
    wx = jnp.maximum(rbx - ltx, 0.0)
    wy = jnp.maximum(rby - lty, 0.0)
    inter = wx * wy
    return inter / (area_g + area_p - inter)


def _fused(pri_ref, bb_ref, gt_ref, out_ref, gbi_ref, pm_s, gbv_s, gbx_s):
    t = pl.program_id(0)
    i = pl.program_id(1)
    nc = pl.num_programs(1)

    @pl.when(t == 0)
    def _pass1():
        iou = _iou_block(pri_ref, bb_ref)

        pmax = jnp.max(iou, axis=0, keepdims=True)
        gidx = lax.broadcasted_iota(jnp.int32, iou.shape, 0)
        pam = jnp.min(jnp.where(iou == pmax, gidx, 128), axis=0, keepdims=True)
        pm_s[i, 0:1, :] = pmax
        pm_s[i, 1:2, :] = pam.astype(jnp.float32)

        cmax = jnp.max(iou, axis=1, keepdims=True)
        pidx = i * _C + lax.broadcasted_iota(jnp.int32, iou.shape, 1)
        cidx = jnp.min(jnp.where(iou == cmax, pidx, 2 ** 30), axis=1,
                       keepdims=True)

        prev_v = jnp.where(i == 0, jnp.float32(-1.0), gbv_s[:, :])
        prev_i = jnp.where(i == 0, jnp.int32(0), gbx_s[:, :])
        upd = cmax > prev_v
        gbv_s[:, :] = jnp.where(upd, cmax, prev_v)
        gbx_s[:, :] = jnp.where(upd, cidx, prev_i)

        @pl.when(i == nc - 1)
        def _():
            gbi_ref[:, :] = gbx_s[:, :]

    @pl.when(t == 1)
    def _pass2():
        pr = pri_ref[:, :]
        pcx, pcy = pr[0:1, :], pr[1:2, :]
        pw, ph = pr[2:3, :], pr[3:4, :]

        pmax = pm_s[i, 0:1, :]
        emid = pm_s[i, 1:2, :].astype(jnp.int32)

        n = bb_ref.shape[0]
        c = pmax.shape[1]
        gidx = lax.broadcasted_iota(jnp.int32, (n, c), 0)

        onehot = (gidx == emid).astype(jnp.float32)
        tx1, ty1 = gt_ref[0:1, :], gt_ref[1:2, :]
        tx2, ty2 = gt_ref[2:3, :], gt_ref[3:4, :]
        labf = gt_ref[4:5, :]
        tab = jnp.concatenate(
            [(tx1 + tx2) / 2, (ty1 + ty2) / 2, tx2 - tx1, ty2 - ty1, labf,
             jnp.zeros((3, n), jnp.float32)], axis=0)
        dims = (((1,), (0,)), ((), ()))
        tab_hi = tab.astype(jnp.bfloat16).astype(jnp.float32)
        tab_lo = tab - tab_hi
        m = (lax.dot_general(tab_hi, onehot, dims,
                             preferred_element_type=jnp.float32)
             + lax.dot_general(tab_lo, onehot, dims,
                               preferred_element_type=jnp.float32))
        mcx, mcy = m[0:1, :], m[1:2, :]
        mw, mh = m[2:3, :], m[3:4, :]
        mlab = m[4:5, :]

        dcx = (mcx - pcx) / pw / 0.1
        dcy = (mcy - pcy) / ph / 0.1
        dw = jnp.log(mw / pw) / 0.2
        dh = jnp.log(mh / ph) / 0.2

        lab = jnp.where(pmax < _POS_T, jnp.float32(-1.0), mlab)
        lab = jnp.where(pmax < _NEG_T, jnp.float32(0.0), lab)

        out_ref[:, :] = jnp.concatenate(
            [dcx, dcy, dw, dh, jnp.round(lab),
             jnp.zeros((3, c), jnp.float32)], axis=0)


def _ln(x):
    bi = lax.bitcast_convert_type(x, jnp.int32)
    e = ((bi >> 23) & 0xFF) - 127
    m = lax.bitcast_convert_type((bi & 0x007FFFFF) | 0x3F800000, jnp.float32)
    s = (m - 1.0) / (m + 1.0)
    s2 = s * s
    ln_m = 2.0 * s * (1.0 + s2 * (1.0 / 3.0 + s2 * (0.2 + s2 * (
        1.0 / 7.0 + s2 * (1.0 / 9.0)))))
    return e.astype(jnp.float32) * 0.6931471805599453 + ln_m


def _sc_patch(npp, outf_hbm, gbi_hbm, gt_hbm, wh_hbm, out_hbm,
              row_v, gbi_v, gt_v, wh_v):
    cid = lax.axis_index("c")
    sid = lax.axis_index("s")

    @pl.when((cid == 0) & (sid == 0))
    def _():
        pltpu.sync_copy(gbi_hbm, gbi_v)
        pltpu.sync_copy(gt_hbm, gt_v)
        pltpu.sync_copy(wh_hbm, wh_v)

        lane = lax.iota(jnp.int32, 16)
        group_vals = []
        group_idx = []
        for v in range(8):
            p = gbi_v[pl.ds(v * 16, 16)]
            base = jnp.full((16,), _LEVEL_BASE[0], jnp.int32)
            stride = jnp.full((16,), _LEVEL_STRIDE[0], jnp.int32)
            shift = jnp.full((16,), _LEVEL_SHIFT[0], jnp.int32)
            for l in range(1, 5):
                sel = p >= _LEVEL_BASE[l]
                base = jnp.where(sel, _LEVEL_BASE[l], base)
                stride = jnp.where(sel, _LEVEL_STRIDE[l], stride)
                shift = jnp.where(sel, _LEVEL_SHIFT[l], shift)
            q = p - base
            cell = q // 9
            s9 = q - cell * 9
            col = cell & ((jnp.int32(1) << shift) - 1)
            row = cell >> shift
            pcx = (col * stride + (stride >> 1)).astype(jnp.float32)
            pcy = (row * stride + (stride >> 1)).astype(jnp.float32)
            widx = (6 - shift) * 9 + s9
            pwv = plsc.load_gather(wh_v, [widx])
            phv = plsc.load_gather(wh_v, [widx + 48])

            gx1 = gt_v[pl.ds(0 * 128 + v * 16, 16)]
            gy1 = gt_v[pl.ds(1 * 128 + v * 16, 16)]
            gx2 = gt_v[pl.ds(2 * 128 + v * 16, 16)]
            gy2 = gt_v[pl.ds(3 * 128 + v * 16, 16)]
            glab = gt_v[pl.ds(4 * 128 + v * 16, 16)]
            gcx = (gx1 + gx2) / 2
            gcy = (gy1 + gy2) / 2
            gw = gx2 - gx1
            gh = gy2 - gy1

            dcx = (gcx - pcx) / pwv / 0.1
            dcy = (gcy - pcy) / phv / 0.1
            dw = _ln(gw / pwv) / 0.2
            dh = _ln(gh / phv) / 0.2
            group_vals.append((dcx, dcy, dw, dh, glab))
            group_idx.append(p)

        for r in range(5):
            pltpu.sync_copy(outf_hbm.at[pl.ds(r * npp, npp)], row_v)
            for v in range(8):
                for l in range(16):
                    plsc.store_scatter(row_v, [group_idx[v]],
                                       group_vals[v][r], mask=lane == l)
            pltpu.sync_copy(row_v, out_hbm.at[pl.ds(r * npp, npp)])


def kernel(image, bboxes, labels, prior_boxes):
    p = prior_boxes.shape[0]
    n = bboxes.shape[0]
    nc = (p + _C - 1) // _C
    pp = nc * _C
    pad_n = pp - p

    pb_t = prior_boxes.T
    pad = jnp.concatenate(
        [jnp.full((2, pad_n), -4096.0, jnp.float32),
         jnp.ones((2, pad_n), jnp.float32)], axis=0)
    pri = jnp.concatenate([pb_t, pad], axis=1)
    pri8 = jnp.concatenate([pri, jnp.zeros((4, pp), jnp.float32)], axis=0)
    gt_t = jnp.concatenate(
        [bboxes.T, labels.astype(jnp.float32)[None, :],
         jnp.zeros((3, n), jnp.float32)], axis=0)

    fp32 = jnp.float32
    outf, gbi = pl.pallas_call(
        _fused,
        grid=(2, nc),
        in_specs=[
            pl.BlockSpec((8, _C), lambda t, i: (0, i)),
            pl.BlockSpec((n, 4), lambda t, i: (0, 0)),
            pl.BlockSpec((8, n), lambda t, i: (0, 0)),
        ],
        out_specs=[
            pl.BlockSpec((8, _C), lambda t, i: (0, i * t)),
            pl.BlockSpec((n, 1), lambda t, i: (0, 0)),
        ],
        out_shape=[
            jax.ShapeDtypeStruct((8, pp), fp32),
            jax.ShapeDtypeStruct((n, 1), jnp.int32),
        ],
        scratch_shapes=[
            pltpu.VMEM((nc, 8, _C), fp32),
            pltpu.VMEM((n, 1), fp32),
            pltpu.VMEM((n, 1), jnp.int32),
        ],
        compiler_params=pltpu.CompilerParams(
            dimension_semantics=("arbitrary", "arbitrary")),
    )(pri8, bboxes, gt_t)

    w45 = jnp.concatenate([prior_boxes[b:b + 9, 2] for b in _LEVEL_BASE])
    h45 = jnp.concatenate([prior_boxes[b:b + 9, 3] for b in _LEVEL_BASE])
    z3 = jnp.zeros((3,), fp32)
    wh = jnp.concatenate([w45, z3, h45, z3])

    mesh = plsc.VectorSubcoreMesh(core_axis_name="c", subcore_axis_name="s")
    patched = pl.kernel(
        functools.partial(_sc_patch, pp),
        mesh=mesh,
        compiler_params=pltpu.CompilerParams(needs_layout_passes=False),
        out_type=jax.ShapeDtypeStruct((5 * pp,), fp32),
        scratch_types=[
            pltpu.VMEM((pp,), fp32),
            pltpu.VMEM((n,), jnp.int32),
            pltpu.VMEM((8 * n,), fp32),
            pltpu.VMEM((96,), fp32),
        ],
    )(outf.reshape(8 * pp)[:5 * pp], gbi.reshape(n), gt_t.reshape(8 * n), wh)

    rows = patched.reshape(5, pp)
    offsets = rows[0:4, :p].T
    lab = rows[4, :p].astype(jnp.int32)
    return (image, offsets, lab)


# speedup vs baseline: 1.8444x; 1.8444x over previous
import jax
import jax.numpy as jnp
from jax import lax
from jax.experimental import pallas as pl
from jax.experimental.pallas import tpu as pltpu

_NEG_T = 0.4
_POS_T = 0.5
_C = 8192


def _iou_block(pri_ref, bb_ref):
    pr = pri_ref[:, :]
    pcx, pcy = pr[0:1, :], pr[1:2, :]
    pw, ph = pr[2:3, :], pr[3:4, :]
    px1 = pcx - pw / 2
    py1 = pcy - ph / 2
    px2 = pcx + pw / 2
    py2 = pcy + ph / 2
    area_p = (px2 - px1) * (py2 - py1)

    gx1, gy1 = bb_ref[:, 0:1], bb_ref[:, 1:2]
    gx2, gy2 = bb_ref[:, 2:3], bb_ref[:, 3:4]
    area_g = (gx2 - gx1) * (gy2 - gy1)

    ltx = jnp.maximum(gx1, px1)
    lty = jnp.maximum(gy1, py1)
    rbx = jnp.minimum(gx2, px2)
    rby = jnp.minimum(gy2, py2)
    wx = jnp.maximum(rbx - ltx, 0.0)
    wy = jnp.maximum(rby - lty, 0.0)
    inter = wx * wy
    return inter / (area_g + area_p - inter)


def _fused(pri_ref, bb_ref, gt_ref, out_ref, pm_s, gbv_s, gbx_s):
    t = pl.program_id(0)
    i = pl.program_id(1)

    @pl.when(t == 0)
    def _pass1():
        iou = _iou_block(pri_ref, bb_ref)

        pmax = jnp.max(iou, axis=0, keepdims=True)
        gidx = lax.broadcasted_iota(jnp.int32, iou.shape, 0)
        pam = jnp.min(jnp.where(iou == pmax, gidx, 128), axis=0, keepdims=True)
        pm_s[i, 0:1, :] = pmax
        pm_s[i, 1:2, :] = pam.astype(jnp.float32)

        cmax = jnp.max(iou, axis=1, keepdims=True)
        pidx = i * _C + lax.broadcasted_iota(jnp.int32, iou.shape, 1)
        cidx = jnp.min(jnp.where(iou == cmax, pidx, 2 ** 30), axis=1,
                       keepdims=True)

        prev_v = jnp.where(i == 0, jnp.float32(-1.0), gbv_s[:, :])
        prev_i = jnp.where(i == 0, jnp.int32(0), gbx_s[:, :])
        upd = cmax > prev_v
        gbv_s[:, :] = jnp.where(upd, cmax, prev_v)
        gbx_s[:, :] = jnp.where(upd, cidx, prev_i)

    @pl.when(t == 1)
    def _pass2():
        pr = pri_ref[:, :]
        pcx, pcy = pr[0:1, :], pr[1:2, :]
        pw, ph = pr[2:3, :], pr[3:4, :]

        pmax = pm_s[i, 0:1, :]
        pam = pm_s[i, 1:2, :].astype(jnp.int32)
        gbi = gbx_s[:, :]

        n = gbi.shape[0]
        c = pmax.shape[1]
        pidx = i * _C + lax.broadcasted_iota(jnp.int32, (n, c), 1)
        gidx = lax.broadcasted_iota(jnp.int32, (n, c), 0)

        forced = jnp.max(jnp.where(gbi == pidx, gidx, -1), axis=0,
                         keepdims=True)
        emid = jnp.where(forced >= 0, forced, pam)
        emax = jnp.where(forced >= 0, jnp.float32(_POS_T), pmax)

        onehot = (gidx == emid).astype(jnp.float32)
        tx1, ty1 = gt_ref[0:1, :], gt_ref[1:2, :]
        tx2, ty2 = gt_ref[2:3, :], gt_ref[3:4, :]
        labf = gt_ref[4:5, :]
        tab = jnp.concatenate(
            [(tx1 + tx2) / 2, (ty1 + ty2) / 2, tx2 - tx1, ty2 - ty1, labf,
             jnp.zeros((3, n), jnp.float32)], axis=0)
        dims = (((1,), (0,)), ((), ()))
        tab_hi = tab.astype(jnp.bfloat16).astype(jnp.float32)
        tab_lo = tab - tab_hi
        m = (lax.dot_general(tab_hi, onehot, dims,
                             preferred_element_type=jnp.float32)
             + lax.dot_general(tab_lo, onehot, dims,
                               preferred_element_type=jnp.float32))
        mcx, mcy = m[0:1, :], m[1:2, :]
        mw, mh = m[2:3, :], m[3:4, :]
        mlab = m[4:5, :]

        dcx = (mcx - pcx) / pw / 0.1
        dcy = (mcy - pcy) / ph / 0.1
        dw = jnp.log(mw / pw) / 0.2
        dh = jnp.log(mh / ph) / 0.2

        lab = jnp.where(emax < _POS_T, jnp.float32(-1.0), mlab)
        lab = jnp.where(emax < _NEG_T, jnp.float32(0.0), lab)

        out_ref[0:5, :] = jnp.concatenate(
            [dcx, dcy, dw, dh, jnp.round(lab)], axis=0)


def kernel(image, bboxes, labels, prior_boxes):
    p = prior_boxes.shape[0]
    n = bboxes.shape[0]
    nc = (p + _C - 1) // _C
    pp = nc * _C
    pad_n = pp - p

    pb_t = prior_boxes.T
    pad = jnp.concatenate(
        [jnp.full((2, pad_n), -4096.0, jnp.float32),
         jnp.ones((2, pad_n), jnp.float32)], axis=0)
    pri = jnp.concatenate([pb_t, pad], axis=1)
    pri8 = jnp.concatenate([pri, jnp.zeros((4, pp), jnp.float32)], axis=0)
    gt_t = jnp.concatenate(
        [bboxes.T, labels.astype(jnp.float32)[None, :],
         jnp.zeros((3, n), jnp.float32)], axis=0)

    fp32 = jnp.float32
    outf = pl.pallas_call(
        _fused,
        grid=(2, nc),
        in_specs=[
            pl.BlockSpec((8, _C), lambda t, i: (0, i)),
            pl.BlockSpec((n, 4), lambda t, i: (0, 0)),
            pl.BlockSpec((8, n), lambda t, i: (0, 0)),
        ],
        out_specs=pl.BlockSpec((8, _C), lambda t, i: (0, i * t)),
        out_shape=jax.ShapeDtypeStruct((8, pp), fp32),
        scratch_shapes=[
            pltpu.VMEM((nc, 8, _C), fp32),
            pltpu.VMEM((n, 1), fp32),
            pltpu.VMEM((n, 1), jnp.int32),
        ],
        compiler_params=pltpu.CompilerParams(
            dimension_semantics=("arbitrary", "arbitrary")),
    )(pri8, bboxes, gt_t)

    offsets = outf[0:4, :p].T
    lab = outf[4, :p].astype(jnp.int32)
    return (image, offsets, lab)


# speedup vs baseline: 2.0930x; 1.1348x over previous
import jax
import jax.numpy as jnp
from jax import lax
from jax.experimental import pallas as pl
from jax.experimental.pallas import tpu as pltpu

_NEG_T = 0.4
_POS_T = 0.5
_C = 8192


def _iou_block(pri_ref, bb_ref):
    pr = pri_ref[:, :]
    pcx, pcy = pr[0:1, :], pr[1:2, :]
    pw, ph = pr[2:3, :], pr[3:4, :]
    px1 = pcx - pw / 2
    py1 = pcy - ph / 2
    px2 = pcx + pw / 2
    py2 = pcy + ph / 2
    area_p = (px2 - px1) * (py2 - py1)

    gx1, gy1 = bb_ref[:, 0:1], bb_ref[:, 1:2]
    gx2, gy2 = bb_ref[:, 2:3], bb_ref[:, 3:4]
    area_g = (gx2 - gx1) * (gy2 - gy1)

    ltx = jnp.maximum(gx1, px1)
    lty = jnp.maximum(gy1, py1)
    rbx = jnp.minimum(gx2, px2)
    rby = jnp.minimum(gy2, py2)
    wx = jnp.maximum(rbx - ltx, 0.0)
    wy = jnp.maximum(rby - lty, 0.0)
    inter = wx * wy
    return inter / (area_g + area_p - inter)


def _fused(pri_ref, bb_ref, gt_ref, out_ref, pm_s, gbv_s, gbx_s):
    t = pl.program_id(0)
    i = pl.program_id(1)

    @pl.when(t == 0)
    def _pass1():
        iou = _iou_block(pri_ref, bb_ref)

        pmax = jnp.max(iou, axis=0, keepdims=True)
        gidx = lax.broadcasted_iota(jnp.int32, iou.shape, 0)
        pam = jnp.min(jnp.where(iou == pmax, gidx, 128), axis=0, keepdims=True)
        pm_s[i, 0:1, :] = pmax
        pm_s[i, 1:2, :] = pam.astype(jnp.float32)

        cmax = jnp.max(iou, axis=1, keepdims=True)
        pidx = i * _C + lax.broadcasted_iota(jnp.int32, iou.shape, 1)
        cidx = jnp.min(jnp.where(iou == cmax, pidx, 2 ** 30), axis=1,
                       keepdims=True)

        prev_v = jnp.where(i == 0, jnp.float32(-1.0), gbv_s[:, :])
        prev_i = jnp.where(i == 0, jnp.int32(0), gbx_s[:, :])
        upd = cmax > prev_v
        gbv_s[:, :] = jnp.where(upd, cmax, prev_v)
        gbx_s[:, :] = jnp.where(upd, cidx, prev_i)

    @pl.when(t == 1)
    def _pass2():
        pr = pri_ref[:, :]
        pcx, pcy = pr[0:1, :], pr[1:2, :]
        pw, ph = pr[2:3, :], pr[3:4, :]

        pmax = pm_s[i, 0:1, :]
        pam = pm_s[i, 1:2, :].astype(jnp.int32)
        gbi = gbx_s[:, :]

        n = gbi.shape[0]
        c = pmax.shape[1]
        pidx = i * _C + lax.broadcasted_iota(jnp.int32, (n, c), 1)
        gidx = lax.broadcasted_iota(jnp.int32, (n, c), 0)

        forced = jnp.max(jnp.where(gbi == pidx, gidx, -1), axis=0,
                         keepdims=True)
        emid = jnp.where(forced >= 0, forced, pam)
        emax = jnp.where(forced >= 0, jnp.float32(_POS_T), pmax)

        onehot = (gidx == emid).astype(jnp.float32)
        tx1, ty1 = gt_ref[0:1, :], gt_ref[1:2, :]
        tx2, ty2 = gt_ref[2:3, :], gt_ref[3:4, :]
        labf = gt_ref[4:5, :]
        tab = jnp.concatenate(
            [(tx1 + tx2) / 2, (ty1 + ty2) / 2, tx2 - tx1, ty2 - ty1, labf,
             jnp.zeros((3, n), jnp.float32)], axis=0)
        dims = (((1,), (0,)), ((), ()))
        tab_hi = tab.astype(jnp.bfloat16).astype(jnp.float32)
        tab_lo = tab - tab_hi
        m = (lax.dot_general(tab_hi, onehot, dims,
                             preferred_element_type=jnp.float32)
             + lax.dot_general(tab_lo, onehot, dims,
                               preferred_element_type=jnp.float32))
        mcx, mcy = m[0:1, :], m[1:2, :]
        mw, mh = m[2:3, :], m[3:4, :]
        mlab = m[4:5, :]

        dcx = (mcx - pcx) / pw / 0.1
        dcy = (mcy - pcy) / ph / 0.1
        dw = jnp.log(mw / pw) / 0.2
        dh = jnp.log(mh / ph) / 0.2

        lab = jnp.where(emax < _POS_T, jnp.float32(-1.0), mlab)
        lab = jnp.where(emax < _NEG_T, jnp.float32(0.0), lab)

        out_ref[0:5, :] = jnp.concatenate(
            [dcx, dcy, dw, dh, jnp.round(lab)], axis=0)


def kernel(image, bboxes, labels, prior_boxes):
    p = prior_boxes.shape[0]
    n = bboxes.shape[0]
    nc = (p + _C - 1) // _C
    pp = nc * _C
    pad_n = pp - p

    pb_t = prior_boxes.T
    pad = jnp.concatenate(
        [jnp.full((2, pad_n), -4096.0, jnp.float32),
         jnp.ones((2, pad_n), jnp.float32)], axis=0)
    pri = jnp.concatenate([pb_t, pad], axis=1)
    pri8 = jnp.concatenate([pri, jnp.zeros((4, pp), jnp.float32)], axis=0)
    gt_t = jnp.concatenate(
        [bboxes.T, labels.astype(jnp.float32)[None, :],
         jnp.zeros((3, n), jnp.float32)], axis=0)

    fp32 = jnp.float32
    outf = pl.pallas_call(
        _fused,
        grid=(2, nc),
        in_specs=[
            pl.BlockSpec((8, _C), lambda t, i: (0, i)),
            pl.BlockSpec((n, 4), lambda t, i: (0, 0)),
            pl.BlockSpec((8, n), lambda t, i: (0, 0)),
        ],
        out_specs=pl.BlockSpec((8, _C), lambda t, i: (0, i * t)),
        out_shape=jax.ShapeDtypeStruct((8, pp), fp32),
        scratch_shapes=[
            pltpu.VMEM((nc, 8, _C), fp32),
            pltpu.VMEM((n, 1), fp32),
            pltpu.VMEM((n, 1), jnp.int32),
        ],
        compiler_params=pltpu.CompilerParams(
            dimension_semantics=("arbitrary", "arbitrary"),
            allow_input_fusion=(True, True, True)),
    )(pri8, bboxes, gt_t)

    offsets = outf[0:4, :p].T
    lab = outf[4, :p].astype(jnp.int32)
    return (image, offsets, lab)


# speedup vs baseline: 2.1094x; 1.0078x over previous
import jax
import jax.numpy as jnp
from jax import lax
from jax.experimental import pallas as pl
from jax.experimental.pallas import tpu as pltpu

_NEG_T = 0.4
_POS_T = 0.5
_C = 16384


def _iou_block(pri_ref, bb_ref):
    pr = pri_ref[:, :]
    pcx, pcy = pr[0:1, :], pr[1:2, :]
    pw, ph = pr[2:3, :], pr[3:4, :]
    px1 = pcx - pw / 2
    py1 = pcy - ph / 2
    px2 = pcx + pw / 2
    py2 = pcy + ph / 2
    area_p = (px2 - px1) * (py2 - py1)

    gx1, gy1 = bb_ref[:, 0:1], bb_ref[:, 1:2]
    gx2, gy2 = bb_ref[:, 2:3], bb_ref[:, 3:4]
    area_g = (gx2 - gx1) * (gy2 - gy1)

    ltx = jnp.maximum(gx1, px1)
    lty = jnp.maximum(gy1, py1)
    rbx = jnp.minimum(gx2, px2)
    rby = jnp.minimum(gy2, py2)
    wx = jnp.maximum(rbx - ltx, 0.0)
    wy = jnp.maximum(rby - lty, 0.0)
    inter = wx * wy
    return inter / (area_g + area_p - inter)


def _fused(pri_ref, bb_ref, gt_ref, out_ref, pm_s, gbv_s, gbx_s):
    t = pl.program_id(0)
    i = pl.program_id(1)

    @pl.when(t == 0)
    def _pass1():
        iou = _iou_block(pri_ref, bb_ref)

        pmax = jnp.max(iou, axis=0, keepdims=True)
        gidx = lax.broadcasted_iota(jnp.int32, iou.shape, 0)
        pam = jnp.min(jnp.where(iou == pmax, gidx, 128), axis=0, keepdims=True)
        pm_s[i, 0:1, :] = pmax
        pm_s[i, 1:2, :] = pam.astype(jnp.float32)

        cmax = jnp.max(iou, axis=1, keepdims=True)
        pidx = i * _C + lax.broadcasted_iota(jnp.int32, iou.shape, 1)
        cidx = jnp.min(jnp.where(iou == cmax, pidx, 2 ** 30), axis=1,
                       keepdims=True)

        prev_v = jnp.where(i == 0, jnp.float32(-1.0), gbv_s[:, :])
        prev_i = jnp.where(i == 0, jnp.int32(0), gbx_s[:, :])
        upd = cmax > prev_v
        gbv_s[:, :] = jnp.where(upd, cmax, prev_v)
        gbx_s[:, :] = jnp.where(upd, cidx, prev_i)

    @pl.when(t == 1)
    def _pass2():
        pr = pri_ref[:, :]
        pcx, pcy = pr[0:1, :], pr[1:2, :]
        pw, ph = pr[2:3, :], pr[3:4, :]

        pmax = pm_s[i, 0:1, :]
        pam = pm_s[i, 1:2, :].astype(jnp.int32)
        gbi = gbx_s[:, :]

        n = gbi.shape[0]
        c = pmax.shape[1]
        pidx = i * _C + lax.broadcasted_iota(jnp.int32, (n, c), 1)
        gidx = lax.broadcasted_iota(jnp.int32, (n, c), 0)

        forced = jnp.max(jnp.where(gbi == pidx, gidx, -1), axis=0,
                         keepdims=True)
        emid = jnp.where(forced >= 0, forced, pam)
        emax = jnp.where(forced >= 0, jnp.float32(_POS_T), pmax)

        onehot = (gidx == emid).astype(jnp.float32)
        tx1, ty1 = gt_ref[0:1, :], gt_ref[1:2, :]
        tx2, ty2 = gt_ref[2:3, :], gt_ref[3:4, :]
        labf = gt_ref[4:5, :]
        tab = jnp.concatenate(
            [(tx1 + tx2) / 2, (ty1 + ty2) / 2, tx2 - tx1, ty2 - ty1, labf,
             jnp.zeros((3, n), jnp.float32)], axis=0)
        dims = (((1,), (0,)), ((), ()))
        tab_hi = tab.astype(jnp.bfloat16).astype(jnp.float32)
        tab_lo = tab - tab_hi
        m = (lax.dot_general(tab_hi, onehot, dims,
                             preferred_element_type=jnp.float32)
             + lax.dot_general(tab_lo, onehot, dims,
                               preferred_element_type=jnp.float32))
        mcx, mcy = m[0:1, :], m[1:2, :]
        mw, mh = m[2:3, :], m[3:4, :]
        mlab = m[4:5, :]

        dcx = (mcx - pcx) / pw / 0.1
        dcy = (mcy - pcy) / ph / 0.1
        dw = jnp.log(mw / pw) / 0.2
        dh = jnp.log(mh / ph) / 0.2

        lab = jnp.where(emax < _POS_T, jnp.float32(-1.0), mlab)
        lab = jnp.where(emax < _NEG_T, jnp.float32(0.0), lab)

        out_ref[0:5, :] = jnp.concatenate(
            [dcx, dcy, dw, dh, jnp.round(lab)], axis=0)


def kernel(image, bboxes, labels, prior_boxes):
    p = prior_boxes.shape[0]
    n = bboxes.shape[0]
    nc = (p + _C - 1) // _C
    pp = nc * _C
    pad_n = pp - p

    pb_t = prior_boxes.T
    pad = jnp.concatenate(
        [jnp.full((2, pad_n), -4096.0, jnp.float32),
         jnp.ones((2, pad_n), jnp.float32)], axis=0)
    pri = jnp.concatenate([pb_t, pad], axis=1)
    pri8 = jnp.concatenate([pri, jnp.zeros((4, pp), jnp.float32)], axis=0)
    gt_t = jnp.concatenate(
        [bboxes.T, labels.astype(jnp.float32)[None, :],
         jnp.zeros((3, n), jnp.float32)], axis=0)

    fp32 = jnp.float32
    outf = pl.pallas_call(
        _fused,
        grid=(2, nc),
        in_specs=[
            pl.BlockSpec((8, _C), lambda t, i: (0, i)),
            pl.BlockSpec((n, 4), lambda t, i: (0, 0)),
            pl.BlockSpec((8, n), lambda t, i: (0, 0)),
        ],
        out_specs=pl.BlockSpec((8, _C), lambda t, i: (0, i * t)),
        out_shape=jax.ShapeDtypeStruct((8, pp), fp32),
        scratch_shapes=[
            pltpu.VMEM((nc, 8, _C), fp32),
            pltpu.VMEM((n, 1), fp32),
            pltpu.VMEM((n, 1), jnp.int32),
        ],
        compiler_params=pltpu.CompilerParams(
            dimension_semantics=("arbitrary", "arbitrary"),
            allow_input_fusion=(True, True, True)),
    )(pri8, bboxes, gt_t)

    offsets = outf[0:4, :p].T
    lab = outf[4, :p].astype(jnp.int32)
    return (image, offsets, lab)
